# baseline (speedup 1.0000x reference)
import functools
import jax
import jax.numpy as jnp
from jax.experimental import pallas as pl

N = 50000
H = 128
BLK = 1024
NPAD = 50176


def _dense_body(agg_ref, cnt_ref, xdst_ref, wl_ref, bl_ref, wr_ref, out_ref):
    inv = 1.0 / jnp.maximum(cnt_ref[...], 1.0)
    mean = agg_ref[...] * inv
    h = jnp.dot(mean, wl_ref[...], preferred_element_type=jnp.float32)
    h += jnp.dot(xdst_ref[...], wr_ref[...], preferred_element_type=jnp.float32)
    out_ref[...] = jnp.maximum(h + bl_ref[...], 0.0)


def _dense2_body(agg_ref, cnt_ref, xdst_ref, wl_ref, bl_ref, wr_ref, w2_ref,
                 b2_ref, out_ref):
    inv = 1.0 / jnp.maximum(cnt_ref[...], 1.0)
    mean = agg_ref[...] * inv
    h = jnp.dot(mean, wl_ref[...], preferred_element_type=jnp.float32)
    h += jnp.dot(xdst_ref[...], wr_ref[...], preferred_element_type=jnp.float32)
    h = jnp.maximum(h + bl_ref[...], 0.0)
    out_ref[...] = jnp.dot(h, w2_ref[...], preferred_element_type=jnp.float32) + b2_ref[...]


def _pad_rows(x):
    return jnp.pad(x, ((0, NPAD - x.shape[0]), (0, 0)))


def _dense(agg, cnt, xdst, WlT, bl, WrT):
    n = agg.shape[0]
    agg = _pad_rows(agg)
    xdst = _pad_rows(xdst)
    cnt = jnp.pad(cnt, (0, NPAD - n))[:, None]
    grid = NPAD // BLK
    out = pl.pallas_call(
        _dense_body,
        grid=(grid,),
        in_specs=[
            pl.BlockSpec((BLK, H), lambda i: (i, 0)),
            pl.BlockSpec((BLK, 1), lambda i: (i, 0)),
            pl.BlockSpec((BLK, H), lambda i: (i, 0)),
            pl.BlockSpec((H, H), lambda i: (0, 0)),
            pl.BlockSpec((1, H), lambda i: (0, 0)),
            pl.BlockSpec((H, H), lambda i: (0, 0)),
        ],
        out_specs=pl.BlockSpec((BLK, H), lambda i: (i, 0)),
        out_shape=jax.ShapeDtypeStruct((NPAD, H), jnp.float32),
    )(agg, cnt, xdst, WlT, bl[None, :], WrT)
    return out[:n]


def _dense2(agg, cnt, xdst, WlT, bl, WrT, W2T, b2):
    n = agg.shape[0]
    agg = _pad_rows(agg)
    xdst = _pad_rows(xdst)
    cnt = jnp.pad(cnt, (0, NPAD - n))[:, None]
    grid = NPAD // BLK
    out = pl.pallas_call(
        _dense2_body,
        grid=(grid,),
        in_specs=[
            pl.BlockSpec((BLK, H), lambda i: (i, 0)),
            pl.BlockSpec((BLK, 1), lambda i: (i, 0)),
            pl.BlockSpec((BLK, H), lambda i: (i, 0)),
            pl.BlockSpec((H, H), lambda i: (0, 0)),
            pl.BlockSpec((1, H), lambda i: (0, 0)),
            pl.BlockSpec((H, H), lambda i: (0, 0)),
            pl.BlockSpec((H, H), lambda i: (0, 0)),
            pl.BlockSpec((1, H), lambda i: (0, 0)),
        ],
        out_specs=pl.BlockSpec((BLK, H), lambda i: (i, 0)),
        out_shape=jax.ShapeDtypeStruct((NPAD, H), jnp.float32),
    )(agg, cnt, xdst, WlT, bl[None, :], WrT, W2T, b2[None, :])
    return out[:n]


def _dec_body(g_ref, w2_ref, b2_ref, out_ref):
    h = jnp.maximum(g_ref[...], 0.0)
    out_ref[...] = jnp.sum(h * w2_ref[...], axis=1, keepdims=True) + b2_ref[...]


def _decoder(g, w2, b2):
    e = g.shape[0]
    epad = ((e + BLK - 1) // BLK) * BLK
    g = jnp.pad(g, ((0, epad - e), (0, 0)))
    out = pl.pallas_call(
        _dec_body,
        grid=(epad // BLK,),
        in_specs=[
            pl.BlockSpec((BLK, H), lambda i: (i, 0)),
            pl.BlockSpec((1, H), lambda i: (0, 0)),
            pl.BlockSpec((1, 1), lambda i: (0, 0)),
        ],
        out_specs=pl.BlockSpec((BLK, 1), lambda i: (i, 0)),
        out_shape=jax.ShapeDtypeStruct((epad, 1), jnp.float32),
    )(g, w2[None, :], b2[None, None])
    return out[:e, 0]


def _segsum(x, ei, n):
    agg = jax.ops.segment_sum(jnp.take(x, ei[0], axis=0), ei[1], num_segments=n)
    cnt = jax.ops.segment_sum(jnp.ones((ei.shape[1],), x.dtype), ei[1],
                              num_segments=n)
    return agg, cnt


def kernel(x_user, x_movie, ei_mm, ei_mu, edge_label_index, user_emb,
           uW1l, ub1, uW1r, uW2l, ub2, uW2r, uW3l, ub3, uW3r, uWlin, ublin,
           mW1l, mb1, mW1r, mW2l, mb2, mW2r, mWlin, mblin,
           dW1, db1, dW2, db2):
    dW1u = dW1[:, :H]
    dW1m = dW1[:, H:]
    Wu = dW1u @ uWlin
    bu = dW1u @ ublin + db1
    Wm = dW1m @ mWlin
    bm = dW1m @ mblin

    agg_mm_x, cnt_mm = _segsum(x_movie, ei_mm, N)
    agg_mu_x, cnt_mu = _segsum(x_movie, ei_mu, N)

    movie_x = _dense(agg_mm_x, cnt_mm, x_movie, uW1l.T, ub1, uW1r.T)
    m1 = _dense(agg_mm_x, cnt_mm, x_movie, mW1l.T, mb1, mW1r.T)
    u1 = _dense(agg_mu_x, cnt_mu, user_emb, uW2l.T, ub2, uW2r.T)

    agg_mu_mx, _ = _segsum(movie_x, ei_mu, N)
    agg_mm_m1, _ = _segsum(m1, ei_mm, N)

    a_user = _dense2(agg_mu_mx, cnt_mu, u1, uW3l.T, ub3, uW3r.T, Wu.T, bu)
    a_movie = _dense2(agg_mm_m1, cnt_mm, m1, mW2l.T, mb2, mW2r.T, Wm.T, bm)

    row = edge_label_index[0]
    col = edge_label_index[1]
    g = jnp.take(a_user, row, axis=0) + jnp.take(a_movie, col, axis=0)
    return _decoder(g, dW2[0], db2[0])


# speedup vs baseline: 3.3457x; 3.3457x over previous
import functools
import jax
import jax.numpy as jnp
from jax import lax
from jax.experimental import pallas as pl
from jax.experimental.pallas import tpu as pltpu
from jax.experimental.pallas import tpu_sc as plsc

N = 50000
H = 128
HALF = 25000
E = 800000
ES = 50000
CR = 2000
CG = 128
RB = 16
ACCR = 26624
BLK = 1024
NPAD = 50176


def _segmean_body(with_cnt, tabL, tabR, srcA, dstA, aggL, aggR, cnt,
                  raw_s, raw_d, csrc, cdst, rows, ones, zb, acc, cacc, sem):
    c = lax.axis_index("c")
    s = lax.axis_index("s")
    lo = c * HALF
    iota = lax.iota(jnp.int32, 16)
    zf = jnp.zeros((16,), jnp.float32)

    def fill1(j, _):
        ones[pl.ds(j * 16, 16)] = jnp.ones((16,), jnp.float32)
        return 0
    lax.fori_loop(0, CG // 16, fill1, 0)

    def fillz(j, _):
        zb[pl.ds(j * 16, 16)] = zf
        return 0
    lax.fori_loop(0, 1664 // 16, fillz, 0)

    def zero_rows():
        def fz(i, _):
            for t in range(4):
                rows[i, pl.ds(t * 16, 16)] = zf
            return 0
        lax.fori_loop(0, CG, fz, 0)

    def zero_acc():
        zero_rows()
        tb = s * 1664
        def fz2(k2, _):
            pltpu.sync_copy(rows, acc.at[pl.ds(tb + k2 * 128, 128)])
            return 0
        lax.fori_loop(0, 13, fz2, 0)

    base = s * ES
    pad_src = c * 256 + s * 16 + iota
    trash = HALF + iota

    for r_i, (tab, agg) in enumerate(((tabL, aggL), (tabR, aggR))):
        zero_acc()
        if with_cnt and r_i == 0:
            pltpu.sync_copy(zb, cacc.at[pl.ds(s * 1664, 1664)])
        plsc.subcore_barrier()
        do_cnt = with_cnt and r_i == 0

        def process(k):
            r = k & (RB - 1)
            pltpu.async_copy(tab.at[csrc.at[r]], rows, sem).wait()
            pltpu.sync_copy(rows, acc.at[cdst.at[r]], add=True)
            if do_cnt:
                pltpu.sync_copy(ones, cacc.at[cdst.at[r]], add=True)

        def comp_chunk(i, off):
            pltpu.sync_copy(srcA.at[pl.ds(base + i * CR, CR)], raw_s)
            pltpu.sync_copy(dstA.at[pl.ds(base + i * CR, CR)], raw_d)

            def inner(j, off):
                sv = raw_s[pl.ds(j * 16, 16)]
                dv = raw_d[pl.ds(j * 16, 16)]
                m = (dv >= lo) & (dv < lo + HALF)
                mi = m.astype(jnp.int32)
                pos = off + plsc.cumsum(mi) - 1
                r = lax.shift_right_arithmetic(pos, 7) & (RB - 1)
                col = pos & (CG - 1)
                plsc.store_scatter(csrc, [r, col], sv, mask=m)
                plsc.store_scatter(cdst, [r, col], dv - lo, mask=m)
                noff = off + jnp.sum(mi)

                @pl.when(lax.shift_right_arithmetic(noff, 7) >
                         lax.shift_right_arithmetic(off, 7))
                def _():
                    process(lax.shift_right_arithmetic(off, 7))
                return noff

            return lax.fori_loop(0, CR // 16, inner, off)

        n = lax.fori_loop(0, ES // CR, comp_chunk, jnp.int32(0))

        @pl.when((n & (CG - 1)) > 0)
        def _():
            npad = lax.shift_left(
                lax.shift_right_arithmetic(n + CG - 1, 7), 7)

            def fpad(t, _):
                pos = n + t * 16 + iota
                m = pos < npad
                r = lax.shift_right_arithmetic(pos, 7) & (RB - 1)
                col = pos & (CG - 1)
                plsc.store_scatter(csrc, [r, col], pad_src, mask=m)
                plsc.store_scatter(cdst, [r, col], trash, mask=m)
                return 0
            lax.fori_loop(0, CG // 16, fpad, 0)
            process(lax.shift_right_arithmetic(n, 7))

        plsc.subcore_barrier()

        def wout(sz, tb):
            pltpu.sync_copy(acc.at[pl.ds(tb, sz)],
                            agg.at[pl.ds(c * HALF + tb, sz)])
            if with_cnt and r_i == 0:
                pltpu.sync_copy(cacc.at[pl.ds(tb, sz)],
                                cnt.at[pl.ds(c * HALF + tb, sz)])

        @pl.when(s < 15)
        def _():
            wout(1568, s * 1568)

        @pl.when(s == 15)
        def _():
            wout(1480, 15 * 1568)

        plsc.subcore_barrier()


def _segmean_sc(tabL, tabR, src, dst, with_cnt):
    mesh = plsc.VectorSubcoreMesh(core_axis_name="c", subcore_axis_name="s",
                                  num_cores=2, num_subcores=16)
    f32 = jnp.float32
    fn = pl.kernel(
        functools.partial(_segmean_body, with_cnt),
        out_type=[jax.ShapeDtypeStruct((N, 64), f32),
                  jax.ShapeDtypeStruct((N, 64), f32),
                  jax.ShapeDtypeStruct((N,), f32)],
        mesh=mesh,
        compiler_params=pltpu.CompilerParams(needs_layout_passes=False, use_tc_tiling_on_sc=False),
        scratch_types=[
            pltpu.VMEM((CR,), jnp.int32),
            pltpu.VMEM((CR,), jnp.int32),
            pltpu.VMEM((RB, CG), jnp.int32),
            pltpu.VMEM((RB, CG), jnp.int32),
            pltpu.VMEM((CG, 64), f32),
            pltpu.VMEM((CG,), f32),
            pltpu.VMEM((1664,), f32),
            pltpu.VMEM_SHARED((ACCR, 64), f32),
            pltpu.VMEM_SHARED((ACCR,), f32),
            pltpu.SemaphoreType.DMA,
        ],
    )
    return fn(tabL, tabR, src, dst)


def _dense_body(aggL_r, aggR_r, cnt_r, xdL_r, xdR_r, wla, wlb, wra, wrb,
                bl_r, outL_r, outR_r):
    inv = 1.0 / jnp.maximum(cnt_r[...], 1.0)
    h = jnp.dot(aggL_r[...] * inv, wla[...], preferred_element_type=jnp.float32)
    h += jnp.dot(aggR_r[...] * inv, wlb[...], preferred_element_type=jnp.float32)
    h += jnp.dot(xdL_r[...], wra[...], preferred_element_type=jnp.float32)
    h += jnp.dot(xdR_r[...], wrb[...], preferred_element_type=jnp.float32)
    h = jnp.maximum(h + bl_r[...], 0.0)
    outL_r[...] = h[:, :64]
    outR_r[...] = h[:, 64:]


def _dense2_body(aggL_r, aggR_r, cnt_r, xdL_r, xdR_r, wla, wlb, wra, wrb,
                 bl_r, w2_r, b2_r, out_r):
    inv = 1.0 / jnp.maximum(cnt_r[...], 1.0)
    h = jnp.dot(aggL_r[...] * inv, wla[...], preferred_element_type=jnp.float32)
    h += jnp.dot(aggR_r[...] * inv, wlb[...], preferred_element_type=jnp.float32)
    h += jnp.dot(xdL_r[...], wra[...], preferred_element_type=jnp.float32)
    h += jnp.dot(xdR_r[...], wrb[...], preferred_element_type=jnp.float32)
    h = jnp.maximum(h + bl_r[...], 0.0)
    out_r[...] = jnp.dot(h, w2_r[...], preferred_element_type=jnp.float32) + b2_r[...]


def _pad_rows(x):
    return jnp.pad(x, ((0, NPAD - x.shape[0]), (0, 0)))


_BS_H = pl.BlockSpec((BLK, 64), lambda i: (i, 0))
_W64 = pl.BlockSpec((64, H), lambda i: (0, 0))
_WFULL = pl.BlockSpec((H, H), lambda i: (0, 0))
_BROW = pl.BlockSpec((1, H), lambda i: (0, 0))


def _dense(aggL, aggR, cnt, xdL, xdR, WlT, bl, WrT):
    ins = [_pad_rows(aggL), _pad_rows(aggR),
           jnp.pad(cnt, (0, NPAD - N))[:, None],
           _pad_rows(xdL), _pad_rows(xdR),
           WlT[:64], WlT[64:], WrT[:64], WrT[64:], bl[None, :]]
    outL, outR = pl.pallas_call(
        _dense_body,
        grid=(NPAD // BLK,),
        in_specs=[_BS_H, _BS_H, pl.BlockSpec((BLK, 1), lambda i: (i, 0)),
                  _BS_H, _BS_H, _W64, _W64, _W64, _W64, _BROW],
        out_specs=[_BS_H, _BS_H],
        out_shape=[jax.ShapeDtypeStruct((NPAD, 64), jnp.float32),
                   jax.ShapeDtypeStruct((NPAD, 64), jnp.float32)],
    )(*ins)
    return outL[:N], outR[:N]


def _dense2(aggL, aggR, cnt, xdL, xdR, WlT, bl, WrT, W2T, b2):
    ins = [_pad_rows(aggL), _pad_rows(aggR),
           jnp.pad(cnt, (0, NPAD - N))[:, None],
           _pad_rows(xdL), _pad_rows(xdR),
           WlT[:64], WlT[64:], WrT[:64], WrT[64:], bl[None, :],
           W2T, b2[None, :]]
    out = pl.pallas_call(
        _dense2_body,
        grid=(NPAD // BLK,),
        in_specs=[_BS_H, _BS_H, pl.BlockSpec((BLK, 1), lambda i: (i, 0)),
                  _BS_H, _BS_H, _W64, _W64, _W64, _W64, _BROW,
                  _WFULL, _BROW],
        out_specs=pl.BlockSpec((BLK, H), lambda i: (i, 0)),
        out_shape=jax.ShapeDtypeStruct((NPAD, H), jnp.float32),
    )(*ins)
    return out[:N]


def _dec_body(g_ref, w2_ref, b2_ref, out_ref):
    h = jnp.maximum(g_ref[...], 0.0)
    out_ref[...] = jnp.sum(h * w2_ref[...], axis=1, keepdims=True) + b2_ref[...]


def _decoder(g, w2, b2):
    e = g.shape[0]
    epad = ((e + BLK - 1) // BLK) * BLK
    g = jnp.pad(g, ((0, epad - e), (0, 0)))
    out = pl.pallas_call(
        _dec_body,
        grid=(epad // BLK,),
        in_specs=[
            pl.BlockSpec((BLK, H), lambda i: (i, 0)),
            pl.BlockSpec((1, H), lambda i: (0, 0)),
            pl.BlockSpec((1, 1), lambda i: (0, 0)),
        ],
        out_specs=pl.BlockSpec((BLK, 1), lambda i: (i, 0)),
        out_shape=jax.ShapeDtypeStruct((epad, 1), jnp.float32),
    )(g, w2[None, :], b2[None, None])
    return out[:e, 0]


def kernel(x_user, x_movie, ei_mm, ei_mu, edge_label_index, user_emb,
           uW1l, ub1, uW1r, uW2l, ub2, uW2r, uW3l, ub3, uW3r, uWlin, ublin,
           mW1l, mb1, mW1r, mW2l, mb2, mW2r, mWlin, mblin,
           dW1, db1, dW2, db2):
    dW1u = dW1[:, :H]
    dW1m = dW1[:, H:]
    Wu = dW1u @ uWlin
    bu = dW1u @ ublin + db1
    Wm = dW1m @ mWlin
    bm = dW1m @ mblin

    xmL = x_movie[:, :64]
    xmR = x_movie[:, 64:]
    ueL = user_emb[:, :64]
    ueR = user_emb[:, 64:]

    aggL_mm, aggR_mm, cnt_mm = _segmean_sc(xmL, xmR, ei_mm[0], ei_mm[1], True)
    aggL_mu, aggR_mu, cnt_mu = _segmean_sc(xmL, xmR, ei_mu[0], ei_mu[1], True)

    mxL, mxR = _dense(aggL_mm, aggR_mm, cnt_mm, xmL, xmR, uW1l.T, ub1, uW1r.T)
    m1L, m1R = _dense(aggL_mm, aggR_mm, cnt_mm, xmL, xmR, mW1l.T, mb1, mW1r.T)
    u1L, u1R = _dense(aggL_mu, aggR_mu, cnt_mu, ueL, ueR, uW2l.T, ub2, uW2r.T)

    aggL_3, aggR_3, _ = _segmean_sc(mxL, mxR, ei_mu[0], ei_mu[1], False)
    aggL_4, aggR_4, _ = _segmean_sc(m1L, m1R, ei_mm[0], ei_mm[1], False)

    a_user = _dense2(aggL_3, aggR_3, cnt_mu, u1L, u1R, uW3l.T, ub3, uW3r.T,
                     Wu.T, bu)
    a_movie = _dense2(aggL_4, aggR_4, cnt_mm, m1L, m1R, mW2l.T, mb2, mW2r.T,
                      Wm.T, bm)

    row = edge_label_index[0]
    col = edge_label_index[1]
    g = jnp.take(a_user, row, axis=0) + jnp.take(a_movie, col, axis=0)
    return _decoder(g, dW2[0], db2[0])


# speedup vs baseline: 4.4664x; 1.3350x over previous
import functools
import jax
import jax.numpy as jnp
from jax import lax
from jax.experimental import pallas as pl
from jax.experimental.pallas import tpu as pltpu
from jax.experimental.pallas import tpu_sc as plsc

N = 50000
H = 128
HALF = 25000
E = 800000
ES = 50000
CR = 2000
CG = 128
RB = 16
ACCR = 25088
BLK = 1024
NPAD = 50176


def _segmean_body(with_cnt, tabL, tabR, srcA, dstA, aggL, aggR, cnt,
                  raw_s, raw_d, csrc, cdst, rows, rows1, ones, zb, acc,
                  cacc, sem, sem1):
    c = lax.axis_index("c")
    s = lax.axis_index("s")
    lo = c * HALF
    iota = lax.iota(jnp.int32, 16)
    zf = jnp.zeros((16,), jnp.float32)

    def fill1(j, _):
        ones[pl.ds(j * 16, 16)] = jnp.ones((16,), jnp.float32)
        return 0
    lax.fori_loop(0, CG // 16, fill1, 0)

    def fillz(j, _):
        zb[pl.ds(j * 16, 16)] = zf
        return 0
    lax.fori_loop(0, 1568 // 16, fillz, 0)

    def zero_rows():
        def fz(i, _):
            for t in range(4):
                rows[i, pl.ds(t * 16, 16)] = zf
            return 0
        lax.fori_loop(0, CG, fz, 0)

    def zero_acc():
        zero_rows()
        tb = s * 1568
        def fz2(k2, _):
            pltpu.sync_copy(rows, acc.at[pl.ds(tb + k2 * 128, 128)])
            return 0
        lax.fori_loop(0, 12, fz2, 0)
        pltpu.sync_copy(rows.at[pl.ds(0, 32)], acc.at[pl.ds(tb + 1536, 32)])

    base = s * ES
    pad_src = c * 256 + s * 16 + iota
    trash = HALF + iota

    for r_i, (tab, agg) in enumerate(((tabL, aggL), (tabR, aggR))):
        zero_acc()
        if with_cnt and r_i == 0:
            pltpu.sync_copy(zb, cacc.at[pl.ds(s * 1568, 1568)])
        plsc.subcore_barrier()
        do_cnt = with_cnt and r_i == 0

        def issue(k):
            r = k & (RB - 1)

            @pl.when((k & 1) == 0)
            def _():
                pltpu.async_copy(tab.at[csrc.at[r]], rows, sem)

            @pl.when((k & 1) == 1)
            def _():
                pltpu.async_copy(tab.at[csrc.at[r]], rows1, sem1)

        def drain(k):
            r = k & (RB - 1)

            @pl.when((k & 1) == 0)
            def _():
                pltpu.make_async_copy(tab.at[csrc.at[r]], rows, sem).wait()
                pltpu.sync_copy(rows, acc.at[cdst.at[r]], add=True)

            @pl.when((k & 1) == 1)
            def _():
                pltpu.make_async_copy(tab.at[csrc.at[r]], rows1, sem1).wait()
                pltpu.sync_copy(rows1, acc.at[cdst.at[r]], add=True)

            if do_cnt:
                pltpu.sync_copy(ones, cacc.at[cdst.at[r]], add=True)

        def process(k):
            issue(k)

            @pl.when(k > 0)
            def _():
                drain(k - 1)

        def comp_chunk(i, off):
            pltpu.sync_copy(srcA.at[pl.ds(base + i * CR, CR)], raw_s)
            pltpu.sync_copy(dstA.at[pl.ds(base + i * CR, CR)], raw_d)

            def inner(j, off):
                sv = raw_s[pl.ds(j * 16, 16)]
                dv = raw_d[pl.ds(j * 16, 16)]
                m = (dv >= lo) & (dv < lo + HALF)
                mi = m.astype(jnp.int32)
                pos = off + plsc.cumsum(mi) - 1
                r = lax.shift_right_arithmetic(pos, 7) & (RB - 1)
                col = pos & (CG - 1)
                plsc.store_scatter(csrc, [r, col], sv, mask=m)
                plsc.store_scatter(cdst, [r, col], dv - lo, mask=m)
                noff = off + jnp.sum(mi)

                @pl.when(lax.shift_right_arithmetic(noff, 7) >
                         lax.shift_right_arithmetic(off, 7))
                def _():
                    process(lax.shift_right_arithmetic(off, 7))
                return noff

            return lax.fori_loop(0, CR // 16, inner, off)

        n = lax.fori_loop(0, ES // CR, comp_chunk, jnp.int32(0))

        nchunks = lax.shift_right_arithmetic(n + CG - 1, 7)

        @pl.when((n & (CG - 1)) > 0)
        def _():
            npad = lax.shift_left(nchunks, 7)

            def fpad(t, _):
                pos = n + t * 16 + iota
                m = pos < npad
                r = lax.shift_right_arithmetic(pos, 7) & (RB - 1)
                col = pos & (CG - 1)
                plsc.store_scatter(csrc, [r, col], pad_src, mask=m)
                plsc.store_scatter(cdst, [r, col], trash, mask=m)
                return 0
            lax.fori_loop(0, CG // 16, fpad, 0)
            process(lax.shift_right_arithmetic(n, 7))

        @pl.when(nchunks > 0)
        def _():
            drain(nchunks - 1)

        plsc.subcore_barrier()

        def wout(sz, tb):
            pltpu.sync_copy(acc.at[pl.ds(tb, sz)],
                            agg.at[pl.ds(c * HALF + tb, sz)])
            if with_cnt and r_i == 0:
                pltpu.sync_copy(cacc.at[pl.ds(tb, sz)],
                                cnt.at[pl.ds(c * HALF + tb, sz)])

        @pl.when(s < 15)
        def _():
            wout(1568, s * 1568)

        @pl.when(s == 15)
        def _():
            wout(1480, 15 * 1568)

        plsc.subcore_barrier()


def _segmean_sc(tabL, tabR, src, dst, with_cnt):
    mesh = plsc.VectorSubcoreMesh(core_axis_name="c", subcore_axis_name="s",
                                  num_cores=2, num_subcores=16)
    f32 = jnp.float32
    fn = pl.kernel(
        functools.partial(_segmean_body, with_cnt),
        out_type=[jax.ShapeDtypeStruct((N, 64), f32),
                  jax.ShapeDtypeStruct((N, 64), f32),
                  jax.ShapeDtypeStruct((N,), f32)],
        mesh=mesh,
        compiler_params=pltpu.CompilerParams(needs_layout_passes=False, use_tc_tiling_on_sc=False),
        scratch_types=[
            pltpu.VMEM((CR,), jnp.int32),
            pltpu.VMEM((CR,), jnp.int32),
            pltpu.VMEM((RB, CG), jnp.int32),
            pltpu.VMEM((RB, CG), jnp.int32),
            pltpu.VMEM((CG, 64), f32),
            pltpu.VMEM((CG, 64), f32),
            pltpu.VMEM((CG,), f32),
            pltpu.VMEM((1568,), f32),
            pltpu.VMEM_SHARED((ACCR, 64), f32),
            pltpu.VMEM_SHARED((ACCR,), f32),
            pltpu.SemaphoreType.DMA,
            pltpu.SemaphoreType.DMA,
        ],
    )
    return fn(tabL, tabR, src, dst)


def _dense_body(aggL_r, aggR_r, cnt_r, xdL_r, xdR_r, wla, wlb, wra, wrb,
                bl_r, outL_r, outR_r):
    inv = 1.0 / jnp.maximum(cnt_r[...], 1.0)
    h = jnp.dot(aggL_r[...] * inv, wla[...], preferred_element_type=jnp.float32)
    h += jnp.dot(aggR_r[...] * inv, wlb[...], preferred_element_type=jnp.float32)
    h += jnp.dot(xdL_r[...], wra[...], preferred_element_type=jnp.float32)
    h += jnp.dot(xdR_r[...], wrb[...], preferred_element_type=jnp.float32)
    h = jnp.maximum(h + bl_r[...], 0.0)
    outL_r[...] = h[:, :64]
    outR_r[...] = h[:, 64:]


def _dense2_body(aggL_r, aggR_r, cnt_r, xdL_r, xdR_r, wla, wlb, wra, wrb,
                 bl_r, w2_r, b2_r, out_r):
    inv = 1.0 / jnp.maximum(cnt_r[...], 1.0)
    h = jnp.dot(aggL_r[...] * inv, wla[...], preferred_element_type=jnp.float32)
    h += jnp.dot(aggR_r[...] * inv, wlb[...], preferred_element_type=jnp.float32)
    h += jnp.dot(xdL_r[...], wra[...], preferred_element_type=jnp.float32)
    h += jnp.dot(xdR_r[...], wrb[...], preferred_element_type=jnp.float32)
    h = jnp.maximum(h + bl_r[...], 0.0)
    out_r[...] = jnp.dot(h, w2_r[...], preferred_element_type=jnp.float32) + b2_r[...]


def _pad_rows(x):
    return jnp.pad(x, ((0, NPAD - x.shape[0]), (0, 0)))


_BS_H = pl.BlockSpec((BLK, 64), lambda i: (i, 0))
_W64 = pl.BlockSpec((64, H), lambda i: (0, 0))
_WFULL = pl.BlockSpec((H, H), lambda i: (0, 0))
_BROW = pl.BlockSpec((1, H), lambda i: (0, 0))


def _dense(aggL, aggR, cnt, xdL, xdR, WlT, bl, WrT):
    ins = [_pad_rows(aggL), _pad_rows(aggR),
           jnp.pad(cnt, (0, NPAD - N))[:, None],
           _pad_rows(xdL), _pad_rows(xdR),
           WlT[:64], WlT[64:], WrT[:64], WrT[64:], bl[None, :]]
    outL, outR = pl.pallas_call(
        _dense_body,
        grid=(NPAD // BLK,),
        in_specs=[_BS_H, _BS_H, pl.BlockSpec((BLK, 1), lambda i: (i, 0)),
                  _BS_H, _BS_H, _W64, _W64, _W64, _W64, _BROW],
        out_specs=[_BS_H, _BS_H],
        out_shape=[jax.ShapeDtypeStruct((NPAD, 64), jnp.float32),
                   jax.ShapeDtypeStruct((NPAD, 64), jnp.float32)],
    )(*ins)
    return outL[:N], outR[:N]


def _dense2(aggL, aggR, cnt, xdL, xdR, WlT, bl, WrT, W2T, b2):
    ins = [_pad_rows(aggL), _pad_rows(aggR),
           jnp.pad(cnt, (0, NPAD - N))[:, None],
           _pad_rows(xdL), _pad_rows(xdR),
           WlT[:64], WlT[64:], WrT[:64], WrT[64:], bl[None, :],
           W2T, b2[None, :]]
    out = pl.pallas_call(
        _dense2_body,
        grid=(NPAD // BLK,),
        in_specs=[_BS_H, _BS_H, pl.BlockSpec((BLK, 1), lambda i: (i, 0)),
                  _BS_H, _BS_H, _W64, _W64, _W64, _W64, _BROW,
                  _WFULL, _BROW],
        out_specs=pl.BlockSpec((BLK, H), lambda i: (i, 0)),
        out_shape=jax.ShapeDtypeStruct((NPAD, H), jnp.float32),
    )(*ins)
    return out[:N]


def _dec_body(g_ref, w2_ref, b2_ref, out_ref):
    h = jnp.maximum(g_ref[...], 0.0)
    out_ref[...] = jnp.sum(h * w2_ref[...], axis=1, keepdims=True) + b2_ref[...]


def _decoder(g, w2, b2):
    e = g.shape[0]
    epad = ((e + BLK - 1) // BLK) * BLK
    g = jnp.pad(g, ((0, epad - e), (0, 0)))
    out = pl.pallas_call(
        _dec_body,
        grid=(epad // BLK,),
        in_specs=[
            pl.BlockSpec((BLK, H), lambda i: (i, 0)),
            pl.BlockSpec((1, H), lambda i: (0, 0)),
            pl.BlockSpec((1, 1), lambda i: (0, 0)),
        ],
        out_specs=pl.BlockSpec((BLK, 1), lambda i: (i, 0)),
        out_shape=jax.ShapeDtypeStruct((epad, 1), jnp.float32),
    )(g, w2[None, :], b2[None, None])
    return out[:e, 0]


def kernel(x_user, x_movie, ei_mm, ei_mu, edge_label_index, user_emb,
           uW1l, ub1, uW1r, uW2l, ub2, uW2r, uW3l, ub3, uW3r, uWlin, ublin,
           mW1l, mb1, mW1r, mW2l, mb2, mW2r, mWlin, mblin,
           dW1, db1, dW2, db2):
    dW1u = dW1[:, :H]
    dW1m = dW1[:, H:]
    Wu = dW1u @ uWlin
    bu = dW1u @ ublin + db1
    Wm = dW1m @ mWlin
    bm = dW1m @ mblin

    xmL = x_movie[:, :64]
    xmR = x_movie[:, 64:]
    ueL = user_emb[:, :64]
    ueR = user_emb[:, 64:]

    aggL_mm, aggR_mm, cnt_mm = _segmean_sc(xmL, xmR, ei_mm[0], ei_mm[1], True)
    aggL_mu, aggR_mu, cnt_mu = _segmean_sc(xmL, xmR, ei_mu[0], ei_mu[1], True)

    mxL, mxR = _dense(aggL_mm, aggR_mm, cnt_mm, xmL, xmR, uW1l.T, ub1, uW1r.T)
    m1L, m1R = _dense(aggL_mm, aggR_mm, cnt_mm, xmL, xmR, mW1l.T, mb1, mW1r.T)
    u1L, u1R = _dense(aggL_mu, aggR_mu, cnt_mu, ueL, ueR, uW2l.T, ub2, uW2r.T)

    aggL_3, aggR_3, _ = _segmean_sc(mxL, mxR, ei_mu[0], ei_mu[1], False)
    aggL_4, aggR_4, _ = _segmean_sc(m1L, m1R, ei_mm[0], ei_mm[1], False)

    a_user = _dense2(aggL_3, aggR_3, cnt_mu, u1L, u1R, uW3l.T, ub3, uW3r.T,
                     Wu.T, bu)
    a_movie = _dense2(aggL_4, aggR_4, cnt_mm, m1L, m1R, mW2l.T, mb2, mW2r.T,
                      Wm.T, bm)

    row = edge_label_index[0]
    col = edge_label_index[1]
    g = jnp.take(a_user, row, axis=0) + jnp.take(a_movie, col, axis=0)
    return _decoder(g, dW2[0], db2[0])


# speedup vs baseline: 4.6143x; 1.0331x over previous
import functools
import jax
import jax.numpy as jnp
from jax import lax
from jax.experimental import pallas as pl
from jax.experimental.pallas import tpu as pltpu
from jax.experimental.pallas import tpu_sc as plsc

N = 50000
H = 128
HALF = 25000
E = 800000
ES = 50000
CR = 2000
CG = 128
RB = 16
ACCR = 25088
BLK = 1024
NPAD = 50176


def _segmean_body(with_cnt, tabL, tabR, srcA, dstA, aggL, aggR, cnt,
                  raw_s, raw_d, csrc, cdst, rows, rows1, ones, zb, acc,
                  cacc, sem, sem1):
    c = lax.axis_index("c")
    s = lax.axis_index("s")
    lo = c * HALF
    iota = lax.iota(jnp.int32, 16)
    zf = jnp.zeros((16,), jnp.float32)

    def fill1(j, _):
        ones[pl.ds(j * 16, 16)] = jnp.ones((16,), jnp.float32)
        return 0
    lax.fori_loop(0, CG // 16, fill1, 0)

    def fillz(j, _):
        zb[pl.ds(j * 16, 16)] = zf
        return 0
    lax.fori_loop(0, 1568 // 16, fillz, 0)

    def zero_rows():
        def fz(i, _):
            for t in range(4):
                rows[i, pl.ds(t * 16, 16)] = zf
            return 0
        lax.fori_loop(0, CG, fz, 0)

    def zero_acc():
        zero_rows()
        tb = s * 1568
        def fz2(k2, _):
            pltpu.sync_copy(rows, acc.at[pl.ds(tb + k2 * 128, 128)])
            return 0
        lax.fori_loop(0, 12, fz2, 0)
        pltpu.sync_copy(rows.at[pl.ds(0, 32)], acc.at[pl.ds(tb + 1536, 32)])

    base = s * ES
    pad_src = c * 256 + s * 16 + iota
    trash = HALF + iota

    for r_i, (tab, agg) in enumerate(((tabL, aggL), (tabR, aggR))):
        zero_acc()
        if with_cnt and r_i == 0:
            pltpu.sync_copy(zb, cacc.at[pl.ds(s * 1568, 1568)])
        plsc.subcore_barrier()
        do_cnt = with_cnt and r_i == 0

        def issue(k):
            r = k & (RB - 1)

            @pl.when((k & 1) == 0)
            def _():
                pltpu.async_copy(tab.at[csrc.at[r]], rows, sem)

            @pl.when((k & 1) == 1)
            def _():
                pltpu.async_copy(tab.at[csrc.at[r]], rows1, sem1)

        def drain(k):
            r = k & (RB - 1)

            @pl.when((k & 1) == 0)
            def _():
                pltpu.make_async_copy(tab.at[csrc.at[r]], rows, sem).wait()
                pltpu.sync_copy(rows, acc.at[cdst.at[r]], add=True)

            @pl.when((k & 1) == 1)
            def _():
                pltpu.make_async_copy(tab.at[csrc.at[r]], rows1, sem1).wait()
                pltpu.sync_copy(rows1, acc.at[cdst.at[r]], add=True)

            if do_cnt:
                pltpu.sync_copy(ones, cacc.at[cdst.at[r]], add=True)

        def process(k):
            issue(k)

            @pl.when(k > 0)
            def _():
                drain(k - 1)

        def comp_chunk(i, off):
            pltpu.sync_copy(srcA.at[pl.ds(base + i * CR, CR)], raw_s)
            pltpu.sync_copy(dstA.at[pl.ds(base + i * CR, CR)], raw_d)

            def inner(j, off):
                sv = raw_s[pl.ds(j * 16, 16)]
                dv = raw_d[pl.ds(j * 16, 16)]
                m = (dv >= lo) & (dv < lo + HALF)
                mi = m.astype(jnp.int32)
                pos = off + plsc.cumsum(mi) - 1
                r = lax.shift_right_arithmetic(pos, 7) & (RB - 1)
                col = pos & (CG - 1)
                plsc.store_scatter(csrc, [r, col], sv, mask=m)
                plsc.store_scatter(cdst, [r, col], dv - lo, mask=m)
                noff = off + jnp.sum(mi)

                @pl.when(lax.shift_right_arithmetic(noff, 7) >
                         lax.shift_right_arithmetic(off, 7))
                def _():
                    process(lax.shift_right_arithmetic(off, 7))
                return noff

            return lax.fori_loop(0, CR // 16, inner, off)

        n = lax.fori_loop(0, ES // CR, comp_chunk, jnp.int32(0))

        nchunks = lax.shift_right_arithmetic(n + CG - 1, 7)

        @pl.when((n & (CG - 1)) > 0)
        def _():
            npad = lax.shift_left(nchunks, 7)

            def fpad(t, _):
                pos = n + t * 16 + iota
                m = pos < npad
                r = lax.shift_right_arithmetic(pos, 7) & (RB - 1)
                col = pos & (CG - 1)
                plsc.store_scatter(csrc, [r, col], pad_src, mask=m)
                plsc.store_scatter(cdst, [r, col], trash, mask=m)
                return 0
            lax.fori_loop(0, CG // 16, fpad, 0)
            process(lax.shift_right_arithmetic(n, 7))

        @pl.when(nchunks > 0)
        def _():
            drain(nchunks - 1)

        plsc.subcore_barrier()

        def wout(sz, tb):
            pltpu.sync_copy(acc.at[pl.ds(tb, sz)],
                            agg.at[pl.ds(c * HALF + tb, sz)])
            if with_cnt and r_i == 0:
                pltpu.sync_copy(cacc.at[pl.ds(tb, sz)],
                                cnt.at[pl.ds(c * HALF + tb, sz)])

        @pl.when(s < 15)
        def _():
            wout(1568, s * 1568)

        @pl.when(s == 15)
        def _():
            wout(1480, 15 * 1568)

        plsc.subcore_barrier()


def _segmean_sc(tabL, tabR, src, dst, with_cnt):
    mesh = plsc.VectorSubcoreMesh(core_axis_name="c", subcore_axis_name="s",
                                  num_cores=2, num_subcores=16)
    f32 = jnp.float32
    fn = pl.kernel(
        functools.partial(_segmean_body, with_cnt),
        out_type=[jax.ShapeDtypeStruct((NPAD, 64), f32),
                  jax.ShapeDtypeStruct((NPAD, 64), f32),
                  jax.ShapeDtypeStruct((NPAD,), f32)],
        mesh=mesh,
        compiler_params=pltpu.CompilerParams(needs_layout_passes=False, use_tc_tiling_on_sc=False),
        scratch_types=[
            pltpu.VMEM((CR,), jnp.int32),
            pltpu.VMEM((CR,), jnp.int32),
            pltpu.VMEM((RB, CG), jnp.int32),
            pltpu.VMEM((RB, CG), jnp.int32),
            pltpu.VMEM((CG, 64), f32),
            pltpu.VMEM((CG, 64), f32),
            pltpu.VMEM((CG,), f32),
            pltpu.VMEM((1568,), f32),
            pltpu.VMEM_SHARED((ACCR, 64), f32),
            pltpu.VMEM_SHARED((ACCR,), f32),
            pltpu.SemaphoreType.DMA,
            pltpu.SemaphoreType.DMA,
        ],
    )
    return fn(tabL, tabR, src, dst)


def _dense_body(aggL_r, aggR_r, cnt_r, xdL_r, xdR_r, wla, wlb, wra, wrb,
                bl_r, outL_r, outR_r):
    inv = 1.0 / jnp.maximum(cnt_r[...], 1.0)
    h = jnp.dot(aggL_r[...] * inv, wla[...], preferred_element_type=jnp.float32)
    h += jnp.dot(aggR_r[...] * inv, wlb[...], preferred_element_type=jnp.float32)
    h += jnp.dot(xdL_r[...], wra[...], preferred_element_type=jnp.float32)
    h += jnp.dot(xdR_r[...], wrb[...], preferred_element_type=jnp.float32)
    h = jnp.maximum(h + bl_r[...], 0.0)
    outL_r[...] = h[:, :64]
    outR_r[...] = h[:, 64:]


def _dense2_body(aggL_r, aggR_r, cnt_r, xdL_r, xdR_r, wla, wlb, wra, wrb,
                 bl_r, w2_r, b2_r, out_r):
    inv = 1.0 / jnp.maximum(cnt_r[...], 1.0)
    h = jnp.dot(aggL_r[...] * inv, wla[...], preferred_element_type=jnp.float32)
    h += jnp.dot(aggR_r[...] * inv, wlb[...], preferred_element_type=jnp.float32)
    h += jnp.dot(xdL_r[...], wra[...], preferred_element_type=jnp.float32)
    h += jnp.dot(xdR_r[...], wrb[...], preferred_element_type=jnp.float32)
    h = jnp.maximum(h + bl_r[...], 0.0)
    out_r[...] = jnp.dot(h, w2_r[...], preferred_element_type=jnp.float32) + b2_r[...]


def _pad_rows(x):
    return jnp.pad(x, ((0, NPAD - x.shape[0]), (0, 0)))


_BS_H = pl.BlockSpec((BLK, 64), lambda i: (i, 0))
_W64 = pl.BlockSpec((64, H), lambda i: (0, 0))
_WFULL = pl.BlockSpec((H, H), lambda i: (0, 0))
_BROW = pl.BlockSpec((1, H), lambda i: (0, 0))


def _dense(aggL, aggR, cnt, xdL, xdR, WlT, bl, WrT):
    ins = [aggL, aggR, cnt[:, None], xdL, xdR,
           WlT[:64], WlT[64:], WrT[:64], WrT[64:], bl[None, :]]
    outL, outR = pl.pallas_call(
        _dense_body,
        grid=(NPAD // BLK,),
        in_specs=[_BS_H, _BS_H, pl.BlockSpec((BLK, 1), lambda i: (i, 0)),
                  _BS_H, _BS_H, _W64, _W64, _W64, _W64, _BROW],
        out_specs=[_BS_H, _BS_H],
        out_shape=[jax.ShapeDtypeStruct((NPAD, 64), jnp.float32),
                   jax.ShapeDtypeStruct((NPAD, 64), jnp.float32)],
    )(*ins)
    return outL, outR


def _dense2(aggL, aggR, cnt, xdL, xdR, WlT, bl, WrT, W2T, b2):
    ins = [aggL, aggR, cnt[:, None], xdL, xdR,
           WlT[:64], WlT[64:], WrT[:64], WrT[64:], bl[None, :],
           W2T, b2[None, :]]
    out = pl.pallas_call(
        _dense2_body,
        grid=(NPAD // BLK,),
        in_specs=[_BS_H, _BS_H, pl.BlockSpec((BLK, 1), lambda i: (i, 0)),
                  _BS_H, _BS_H, _W64, _W64, _W64, _W64, _BROW,
                  _WFULL, _BROW],
        out_specs=pl.BlockSpec((BLK, H), lambda i: (i, 0)),
        out_shape=jax.ShapeDtypeStruct((NPAD, H), jnp.float32),
    )(*ins)
    return out


def _dec_body(g_ref, w2_ref, b2_ref, out_ref):
    h = jnp.maximum(g_ref[...], 0.0)
    out_ref[...] = jnp.sum(h * w2_ref[...], axis=1, keepdims=True) + b2_ref[...]


def _decoder(g, w2, b2):
    e = g.shape[0]
    epad = ((e + BLK - 1) // BLK) * BLK
    g = jnp.pad(g, ((0, epad - e), (0, 0)))
    out = pl.pallas_call(
        _dec_body,
        grid=(epad // BLK,),
        in_specs=[
            pl.BlockSpec((BLK, H), lambda i: (i, 0)),
            pl.BlockSpec((1, H), lambda i: (0, 0)),
            pl.BlockSpec((1, 1), lambda i: (0, 0)),
        ],
        out_specs=pl.BlockSpec((BLK, 1), lambda i: (i, 0)),
        out_shape=jax.ShapeDtypeStruct((epad, 1), jnp.float32),
    )(g, w2[None, :], b2[None, None])
    return out[:e, 0]


def kernel(x_user, x_movie, ei_mm, ei_mu, edge_label_index, user_emb,
           uW1l, ub1, uW1r, uW2l, ub2, uW2r, uW3l, ub3, uW3r, uWlin, ublin,
           mW1l, mb1, mW1r, mW2l, mb2, mW2r, mWlin, mblin,
           dW1, db1, dW2, db2):
    dW1u = dW1[:, :H]
    dW1m = dW1[:, H:]
    Wu = dW1u @ uWlin
    bu = dW1u @ ublin + db1
    Wm = dW1m @ mWlin
    bm = dW1m @ mblin

    xmL = _pad_rows(x_movie[:, :64])
    xmR = _pad_rows(x_movie[:, 64:])
    ueL = _pad_rows(user_emb[:, :64])
    ueR = _pad_rows(user_emb[:, 64:])

    aggL_mm, aggR_mm, cnt_mm = _segmean_sc(xmL, xmR, ei_mm[0], ei_mm[1], True)
    aggL_mu, aggR_mu, cnt_mu = _segmean_sc(xmL, xmR, ei_mu[0], ei_mu[1], True)

    mxL, mxR = _dense(aggL_mm, aggR_mm, cnt_mm, xmL, xmR, uW1l.T, ub1, uW1r.T)
    m1L, m1R = _dense(aggL_mm, aggR_mm, cnt_mm, xmL, xmR, mW1l.T, mb1, mW1r.T)
    u1L, u1R = _dense(aggL_mu, aggR_mu, cnt_mu, ueL, ueR, uW2l.T, ub2, uW2r.T)

    aggL_3, aggR_3, _ = _segmean_sc(mxL, mxR, ei_mu[0], ei_mu[1], False)
    aggL_4, aggR_4, _ = _segmean_sc(m1L, m1R, ei_mm[0], ei_mm[1], False)

    a_user = _dense2(aggL_3, aggR_3, cnt_mu, u1L, u1R, uW3l.T, ub3, uW3r.T,
                     Wu.T, bu)
    a_movie = _dense2(aggL_4, aggR_4, cnt_mm, m1L, m1R, mW2l.T, mb2, mW2r.T,
                      Wm.T, bm)

    row = edge_label_index[0]
    col = edge_label_index[1]
    g = jnp.take(a_user, row, axis=0) + jnp.take(a_movie, col, axis=0)
    return _decoder(g, dW2[0], db2[0])


# speedup vs baseline: 5.6334x; 1.2209x over previous
import functools
import jax
import jax.numpy as jnp
from jax import lax
from jax.experimental import pallas as pl
from jax.experimental.pallas import tpu as pltpu
from jax.experimental.pallas import tpu_sc as plsc

N = 50000
H = 128
HALF = 25000
E = 800000
ES = 50000
CR = 2000
CG = 128
RB = 16
ACCR = 25088
BLK = 1024
NPAD = 50176
ELBL = 200000
ELP = 200704
LW = 6272


def _segmean_body(with_cnt, tabL, tabR, srcA, dstA, aggL, aggR, cnt,
                  raw_s, raw_d, csrc, cdst, rows, rows1, ones, zb, acc,
                  cacc, sem, sem1):
    c = lax.axis_index("c")
    s = lax.axis_index("s")
    lo = c * HALF
    iota = lax.iota(jnp.int32, 16)
    zf = jnp.zeros((16,), jnp.float32)

    def fill1(j, _):
        ones[pl.ds(j * 16, 16)] = jnp.ones((16,), jnp.float32)
        return 0
    lax.fori_loop(0, CG // 16, fill1, 0)

    def fillz(j, _):
        zb[pl.ds(j * 16, 16)] = zf
        return 0
    lax.fori_loop(0, 1568 // 16, fillz, 0)

    def zero_rows():
        def fz(i, _):
            for t in range(4):
                rows[i, pl.ds(t * 16, 16)] = zf
            return 0
        lax.fori_loop(0, CG, fz, 0)

    def zero_acc():
        zero_rows()
        tb = s * 1568
        def fz2(k2, _):
            pltpu.sync_copy(rows, acc.at[pl.ds(tb + k2 * 128, 128)])
            return 0
        lax.fori_loop(0, 12, fz2, 0)
        pltpu.sync_copy(rows.at[pl.ds(0, 32)], acc.at[pl.ds(tb + 1536, 32)])

    base = s * ES
    pad_src = c * 256 + s * 16 + iota
    trash = HALF + iota

    for r_i, (tab, agg) in enumerate(((tabL, aggL), (tabR, aggR))):
        zero_acc()
        if with_cnt and r_i == 0:
            pltpu.sync_copy(zb, cacc.at[pl.ds(s * 1568, 1568)])
        plsc.subcore_barrier()
        do_cnt = with_cnt and r_i == 0

        def issue(k):
            r = k & (RB - 1)

            @pl.when((k & 1) == 0)
            def _():
                pltpu.async_copy(tab.at[csrc.at[r]], rows, sem)

            @pl.when((k & 1) == 1)
            def _():
                pltpu.async_copy(tab.at[csrc.at[r]], rows1, sem1)

        def drain(k):
            r = k & (RB - 1)

            @pl.when((k & 1) == 0)
            def _():
                pltpu.make_async_copy(tab.at[csrc.at[r]], rows, sem).wait()
                pltpu.sync_copy(rows, acc.at[cdst.at[r]], add=True)

            @pl.when((k & 1) == 1)
            def _():
                pltpu.make_async_copy(tab.at[csrc.at[r]], rows1, sem1).wait()
                pltpu.sync_copy(rows1, acc.at[cdst.at[r]], add=True)

            if do_cnt:
                pltpu.sync_copy(ones, cacc.at[cdst.at[r]], add=True)

        def process(k):
            issue(k)

            @pl.when(k > 0)
            def _():
                drain(k - 1)

        def comp_chunk(i, off):
            pltpu.sync_copy(srcA.at[pl.ds(base + i * CR, CR)], raw_s)
            pltpu.sync_copy(dstA.at[pl.ds(base + i * CR, CR)], raw_d)

            def inner(j, off):
                sv = raw_s[pl.ds(j * 16, 16)]
                dv = raw_d[pl.ds(j * 16, 16)]
                m = (dv >= lo) & (dv < lo + HALF)
                mi = m.astype(jnp.int32)
                pos = off + plsc.cumsum(mi) - 1
                r = lax.shift_right_arithmetic(pos, 7) & (RB - 1)
                col = pos & (CG - 1)
                plsc.store_scatter(csrc, [r, col], sv, mask=m)
                plsc.store_scatter(cdst, [r, col], dv - lo, mask=m)
                noff = off + jnp.sum(mi)

                @pl.when(lax.shift_right_arithmetic(noff, 7) >
                         lax.shift_right_arithmetic(off, 7))
                def _():
                    process(lax.shift_right_arithmetic(off, 7))
                return noff

            return lax.fori_loop(0, CR // 16, inner, off)

        n = lax.fori_loop(0, ES // CR, comp_chunk, jnp.int32(0))

        nchunks = lax.shift_right_arithmetic(n + CG - 1, 7)

        @pl.when((n & (CG - 1)) > 0)
        def _():
            npad = lax.shift_left(nchunks, 7)

            def fpad(t, _):
                pos = n + t * 16 + iota
                m = pos < npad
                r = lax.shift_right_arithmetic(pos, 7) & (RB - 1)
                col = pos & (CG - 1)
                plsc.store_scatter(csrc, [r, col], pad_src, mask=m)
                plsc.store_scatter(cdst, [r, col], trash, mask=m)
                return 0
            lax.fori_loop(0, CG // 16, fpad, 0)
            process(lax.shift_right_arithmetic(n, 7))

        @pl.when(nchunks > 0)
        def _():
            drain(nchunks - 1)

        plsc.subcore_barrier()

        def wout(sz, tb):
            pltpu.sync_copy(acc.at[pl.ds(tb, sz)],
                            agg.at[pl.ds(c * HALF + tb, sz)])
            if with_cnt and r_i == 0:
                pltpu.sync_copy(cacc.at[pl.ds(tb, sz)],
                                cnt.at[pl.ds(c * HALF + tb, sz)])

        @pl.when(s < 15)
        def _():
            wout(1568, s * 1568)

        @pl.when(s == 15)
        def _():
            wout(1480, 15 * 1568)

        plsc.subcore_barrier()


def _segmean_sc(tabL, tabR, src, dst, with_cnt):
    mesh = plsc.VectorSubcoreMesh(core_axis_name="c", subcore_axis_name="s",
                                  num_cores=2, num_subcores=16)
    f32 = jnp.float32
    fn = pl.kernel(
        functools.partial(_segmean_body, with_cnt),
        out_type=[jax.ShapeDtypeStruct((NPAD, 64), f32),
                  jax.ShapeDtypeStruct((NPAD, 64), f32),
                  jax.ShapeDtypeStruct((NPAD,), f32)],
        mesh=mesh,
        compiler_params=pltpu.CompilerParams(needs_layout_passes=False, use_tc_tiling_on_sc=False),
        scratch_types=[
            pltpu.VMEM((CR,), jnp.int32),
            pltpu.VMEM((CR,), jnp.int32),
            pltpu.VMEM((RB, CG), jnp.int32),
            pltpu.VMEM((RB, CG), jnp.int32),
            pltpu.VMEM((CG, 64), f32),
            pltpu.VMEM((CG, 64), f32),
            pltpu.VMEM((CG,), f32),
            pltpu.VMEM((1568,), f32),
            pltpu.VMEM_SHARED((ACCR, 64), f32),
            pltpu.VMEM_SHARED((ACCR,), f32),
            pltpu.SemaphoreType.DMA,
            pltpu.SemaphoreType.DMA,
        ],
    )
    return fn(tabL, tabR, src, dst)


def _dense_body(aggL_r, aggR_r, cnt_r, xdL_r, xdR_r, wla, wlb, wra, wrb,
                bl_r, outL_r, outR_r):
    inv = 1.0 / jnp.maximum(cnt_r[...], 1.0)
    h = jnp.dot(aggL_r[...] * inv, wla[...], preferred_element_type=jnp.float32)
    h += jnp.dot(aggR_r[...] * inv, wlb[...], preferred_element_type=jnp.float32)
    h += jnp.dot(xdL_r[...], wra[...], preferred_element_type=jnp.float32)
    h += jnp.dot(xdR_r[...], wrb[...], preferred_element_type=jnp.float32)
    h = jnp.maximum(h + bl_r[...], 0.0)
    outL_r[...] = h[:, :64]
    outR_r[...] = h[:, 64:]


def _dense2_body(aggL_r, aggR_r, cnt_r, xdL_r, xdR_r, wla, wlb, wra, wrb,
                 bl_r, w2_r, b2_r, out_r):
    inv = 1.0 / jnp.maximum(cnt_r[...], 1.0)
    h = jnp.dot(aggL_r[...] * inv, wla[...], preferred_element_type=jnp.float32)
    h += jnp.dot(aggR_r[...] * inv, wlb[...], preferred_element_type=jnp.float32)
    h += jnp.dot(xdL_r[...], wra[...], preferred_element_type=jnp.float32)
    h += jnp.dot(xdR_r[...], wrb[...], preferred_element_type=jnp.float32)
    h = jnp.maximum(h + bl_r[...], 0.0)
    out_r[...] = jnp.dot(h, w2_r[...], preferred_element_type=jnp.float32) + b2_r[...]


def _pad_rows(x):
    return jnp.pad(x, ((0, NPAD - x.shape[0]), (0, 0)))


_BS_H = pl.BlockSpec((BLK, 64), lambda i: (i, 0))
_W64 = pl.BlockSpec((64, H), lambda i: (0, 0))
_WFULL = pl.BlockSpec((H, H), lambda i: (0, 0))
_BROW = pl.BlockSpec((1, H), lambda i: (0, 0))


def _dense(aggL, aggR, cnt, xdL, xdR, WlT, bl, WrT):
    ins = [aggL, aggR, cnt[:, None], xdL, xdR,
           WlT[:64], WlT[64:], WrT[:64], WrT[64:], bl[None, :]]
    outL, outR = pl.pallas_call(
        _dense_body,
        grid=(NPAD // BLK,),
        in_specs=[_BS_H, _BS_H, pl.BlockSpec((BLK, 1), lambda i: (i, 0)),
                  _BS_H, _BS_H, _W64, _W64, _W64, _W64, _BROW],
        out_specs=[_BS_H, _BS_H],
        out_shape=[jax.ShapeDtypeStruct((NPAD, 64), jnp.float32),
                   jax.ShapeDtypeStruct((NPAD, 64), jnp.float32)],
    )(*ins)
    return outL, outR


def _dense2(aggL, aggR, cnt, xdL, xdR, WlT, bl, WrT, W2T, b2):
    ins = [aggL, aggR, cnt[:, None], xdL, xdR,
           WlT[:64], WlT[64:], WrT[:64], WrT[64:], bl[None, :],
           W2T, b2[None, :]]
    out = pl.pallas_call(
        _dense2_body,
        grid=(NPAD // BLK,),
        in_specs=[_BS_H, _BS_H, pl.BlockSpec((BLK, 1), lambda i: (i, 0)),
                  _BS_H, _BS_H, _W64, _W64, _W64, _W64, _BROW,
                  _WFULL, _BROW],
        out_specs=pl.BlockSpec((BLK, H), lambda i: (i, 0)),
        out_shape=jax.ShapeDtypeStruct((NPAD, H), jnp.float32),
    )(*ins)
    return out


def _lgather_body(au, am, rowA, colA, g, idxb, buf, sem):
    c = lax.axis_index("c")
    s = lax.axis_index("s")
    base = (c * 16 + s) * LW

    def chunk(k, _):
        pltpu.sync_copy(rowA.at[pl.ds(base + k * 128, 128)], idxb)
        pltpu.async_copy(au.at[idxb], buf, sem).wait()
        pltpu.sync_copy(colA.at[pl.ds(base + k * 128, 128)], idxb)
        pltpu.async_copy(am.at[idxb], buf, sem, add=True).wait()
        pltpu.sync_copy(buf, g.at[pl.ds(base + k * 128, 128)])
        return 0
    lax.fori_loop(0, LW // 128, chunk, 0)


def _lgather_sc(au, am, row, col):
    mesh = plsc.VectorSubcoreMesh(core_axis_name="c", subcore_axis_name="s",
                                  num_cores=2, num_subcores=16)
    fn = pl.kernel(
        _lgather_body,
        out_type=jax.ShapeDtypeStruct((ELP, H), jnp.float32),
        mesh=mesh,
        compiler_params=pltpu.CompilerParams(needs_layout_passes=False,
                                             use_tc_tiling_on_sc=False),
        scratch_types=[
            pltpu.VMEM((128,), jnp.int32),
            pltpu.VMEM((128, H), jnp.float32),
            pltpu.SemaphoreType.DMA,
        ],
    )
    return fn(au, am, jnp.pad(row, (0, ELP - ELBL)),
              jnp.pad(col, (0, ELP - ELBL)))


def _dec_body(g_ref, w2_ref, b2_ref, out_ref):
    h = jnp.maximum(g_ref[...], 0.0)
    out_ref[...] = jnp.sum(h * w2_ref[...], axis=1, keepdims=True) + b2_ref[...]


def _decoder(g, w2, b2):
    e = g.shape[0]
    epad = ((e + BLK - 1) // BLK) * BLK
    g = jnp.pad(g, ((0, epad - e), (0, 0)))
    out = pl.pallas_call(
        _dec_body,
        grid=(epad // BLK,),
        in_specs=[
            pl.BlockSpec((BLK, H), lambda i: (i, 0)),
            pl.BlockSpec((1, H), lambda i: (0, 0)),
            pl.BlockSpec((1, 1), lambda i: (0, 0)),
        ],
        out_specs=pl.BlockSpec((BLK, 1), lambda i: (i, 0)),
        out_shape=jax.ShapeDtypeStruct((epad, 1), jnp.float32),
    )(g, w2[None, :], b2[None, None])
    return out[:e, 0]


def kernel(x_user, x_movie, ei_mm, ei_mu, edge_label_index, user_emb,
           uW1l, ub1, uW1r, uW2l, ub2, uW2r, uW3l, ub3, uW3r, uWlin, ublin,
           mW1l, mb1, mW1r, mW2l, mb2, mW2r, mWlin, mblin,
           dW1, db1, dW2, db2):
    dW1u = dW1[:, :H]
    dW1m = dW1[:, H:]
    Wu = dW1u @ uWlin
    bu = dW1u @ ublin + db1
    Wm = dW1m @ mWlin
    bm = dW1m @ mblin

    xmL = _pad_rows(x_movie[:, :64])
    xmR = _pad_rows(x_movie[:, 64:])
    ueL = _pad_rows(user_emb[:, :64])
    ueR = _pad_rows(user_emb[:, 64:])

    aggL_mm, aggR_mm, cnt_mm = _segmean_sc(xmL, xmR, ei_mm[0], ei_mm[1], True)
    aggL_mu, aggR_mu, cnt_mu = _segmean_sc(xmL, xmR, ei_mu[0], ei_mu[1], True)

    mxL, mxR = _dense(aggL_mm, aggR_mm, cnt_mm, xmL, xmR, uW1l.T, ub1, uW1r.T)
    m1L, m1R = _dense(aggL_mm, aggR_mm, cnt_mm, xmL, xmR, mW1l.T, mb1, mW1r.T)
    u1L, u1R = _dense(aggL_mu, aggR_mu, cnt_mu, ueL, ueR, uW2l.T, ub2, uW2r.T)

    aggL_3, aggR_3, _ = _segmean_sc(mxL, mxR, ei_mu[0], ei_mu[1], False)
    aggL_4, aggR_4, _ = _segmean_sc(m1L, m1R, ei_mm[0], ei_mm[1], False)

    a_user = _dense2(aggL_3, aggR_3, cnt_mu, u1L, u1R, uW3l.T, ub3, uW3r.T,
                     Wu.T, bu)
    a_movie = _dense2(aggL_4, aggR_4, cnt_mm, m1L, m1R, mW2l.T, mb2, mW2r.T,
                      Wm.T, bm)

    g = _lgather_sc(a_user, a_movie, edge_label_index[0],
                    edge_label_index[1])
    return _decoder(g, dW2[0], db2[0])[:ELBL]
